# baseline (speedup 1.0000x reference)
import functools

import jax
import jax.numpy as jnp
from jax import lax
from jax.experimental import pallas as pl
from jax.experimental.pallas import tpu as pltpu
from jax.experimental.pallas import tpu_sc as plsc

N_ATOMS = 10000
BATCH = 8
ROW = 16
NC, NS = 2, 16
GRP = 128
G = 160
EPW = G * GRP
E_PAD = NS * EPW
N_PAD = 10112
STRIPE = N_PAD // NS
NBUF = 16
N_ITER = 2


def _mp_sc(h_hbm, src_hbm, dst_hbm, zeros_hbm, tv_hbm, ix_hbm, out_hbm, *scratch):
    src_v, dst_v = scratch[0], scratch[1]
    rows = scratch[2:2 + NBUF]
    acc_v, tbl_v, tv_v, ix_v = scratch[2 + NBUF:6 + NBUF]
    tbl_s, acc_s = scratch[6 + NBUF], scratch[7 + NBUF]
    sems = scratch[8 + NBUF:]
    c = lax.axis_index("c")
    s = lax.axis_index("s")
    stripe = pl.ds(s * STRIPE, STRIPE)
    pltpu.sync_copy(src_hbm.at[s], src_v)
    pltpu.sync_copy(dst_hbm.at[s], dst_v)
    pltpu.sync_copy(tv_hbm, tv_v)
    pltpu.sync_copy(ix_hbm, ix_v)
    pltpu.sync_copy(h_hbm.at[c, stripe], tbl_v)
    pltpu.sync_copy(tbl_v, tbl_s.at[stripe])

    def gather(g, b):
        return pltpu.make_async_copy(tbl_s.at[src_v.at[g]], rows[b], sems[b])

    for t in range(N_ITER):
        pltpu.sync_copy(zeros_hbm.at[stripe], acc_v)
        pltpu.sync_copy(acc_v, acc_s.at[stripe])
        plsc.subcore_barrier()

        for b in range(NBUF):
            gather(b, b).start()

        def body(i, _):
            for b in range(NBUF):
                g = i * NBUF + b
                gather(g, b).wait()
                pltpu.sync_copy(rows[b], acc_s.at[dst_v.at[g]], add=True)

                @pl.when(g < G - NBUF)
                def _fire():
                    gather(g + NBUF, b).start()
            return _

        lax.fori_loop(0, G // NBUF, body, None)
        plsc.subcore_barrier()

        pltpu.sync_copy(acc_s.at[stripe], acc_v)
        pltpu.sync_copy(tbl_s.at[stripe], tbl_v)
        tv0, tv1, tv2, tv3 = tv_v[0], tv_v[1], tv_v[2], tv_v[3]
        bvec = tv_v[4]

        _dn = lax.GatherDimensionNumbers(
            offset_dims=(), collapsed_slice_dims=(0,), start_index_map=(0,))

        def take16(x, idx):
            return lax.gather(x, idx.reshape(16, 1), _dn, slice_sizes=(1,),
                              mode=lax.GatherScatterMode.PROMISE_IN_BOUNDS)

        b0, b1, b2, b3 = ix_v[0], ix_v[1], ix_v[2], ix_v[3]

        def upd(r, _):
            S = acc_v[r]
            gv0 = take16(S, b0)
            gv1 = take16(S, b1)
            gv2 = take16(S, b2)
            gv3 = take16(S, b3)
            q = 1.0 / jnp.maximum(gv3, 1.0)
            mixed = gv0 * tv0 + gv1 * tv1 + gv2 * tv2 + gv3 * tv3
            tbl_v[r] = tbl_v[r] + q * mixed + bvec
            return _

        lax.fori_loop(0, STRIPE, upd, None)
        if t < N_ITER - 1:
            pltpu.sync_copy(tbl_v, tbl_s.at[stripe])
            plsc.subcore_barrier()

    pltpu.sync_copy(tbl_v, out_hbm.at[c, stripe])


@functools.cache
def _build_mp():
    return pl.kernel(
        _mp_sc,
        mesh=plsc.VectorSubcoreMesh(core_axis_name="c", subcore_axis_name="s"),
        compiler_params=pltpu.CompilerParams(use_tc_tiling_on_sc=False),
        out_type=jax.ShapeDtypeStruct((NC, N_PAD, ROW), jnp.float32),
        scratch_types=(
            [pltpu.VMEM((G, GRP), jnp.int32),
             pltpu.VMEM((G, GRP), jnp.int32)]
            + [pltpu.VMEM((GRP, ROW), jnp.float32) for _ in range(NBUF)]
            + [pltpu.VMEM((STRIPE, ROW), jnp.float32),
               pltpu.VMEM((STRIPE, ROW), jnp.float32),
               pltpu.VMEM((8, 16), jnp.float32),
               pltpu.VMEM((8, 16), jnp.int32),
               pltpu.VMEM_SHARED((N_PAD, ROW), jnp.float32),
               pltpu.VMEM_SHARED((N_PAD, ROW), jnp.float32)]
            + [pltpu.SemaphoreType.DMA for _ in range(NBUF)]
        ),
    )


def kernel(positions, bonds, W_msg, b_msg, W_upd, b_upd):
    src = jnp.concatenate([bonds[:, 0], bonds[:, 1]])
    dst = jnp.concatenate([bonds[:, 1], bonds[:, 0]])
    pad = jnp.full((E_PAD - src.shape[0],), N_ATOMS, jnp.int32)
    src = jnp.concatenate([src, pad]).reshape(NS, G, GRP)
    dst = jnp.concatenate([dst, pad]).reshape(NS, G, GRP)

    M = W_upd @ W_msg
    v = W_upd @ b_msg
    t4 = jnp.zeros((4, 4), jnp.float32).at[:3, :3].set(M.T).at[3, :3].set(v)
    tv = jnp.zeros((8, 16), jnp.float32)
    tv = tv.at[:4, :].set(jnp.tile(t4, (1, 4)))
    tv = tv.at[4, :].set(jnp.tile(
        jnp.concatenate([b_upd, jnp.zeros((1,), jnp.float32)]), 4))

    hp = jnp.concatenate(
        [positions.transpose(1, 0, 2),
         jnp.ones((N_ATOMS, BATCH, 1), jnp.float32)], axis=2
    ).reshape(N_ATOMS, BATCH, 4)
    h = jnp.zeros((NC, N_PAD, ROW), jnp.float32)
    h = h.at[0, :N_ATOMS].set(hp[:, :4].reshape(N_ATOMS, ROW))
    h = h.at[1, :N_ATOMS].set(hp[:, 4:].reshape(N_ATOMS, ROW))

    zeros = jnp.zeros((N_PAD, ROW), jnp.float32)
    lane = jnp.arange(16, dtype=jnp.int32)
    ix = jnp.zeros((8, 16), jnp.int32)
    for cc in range(4):
        ix = ix.at[cc].set((lane // 4) * 4 + cc)
    out = _build_mp()(h, src, dst, zeros, tv, ix)

    o = out[:, :N_ATOMS].reshape(NC, N_ATOMS, 4, 4)[:, :, :, :3]
    return o.transpose(0, 2, 1, 3).reshape(BATCH, N_ATOMS, 3)


# speedup vs baseline: 1.0050x; 1.0050x over previous
import functools

import jax
import jax.numpy as jnp
from jax import lax
from jax.experimental import pallas as pl
from jax.experimental.pallas import tpu as pltpu
from jax.experimental.pallas import tpu_sc as plsc

N_ATOMS = 10000
BATCH = 8
ROW = 16
NC, NS = 2, 16
GRP = 128
G = 160
EPW = G * GRP
E_PAD = NS * EPW
N_PAD = 10112
STRIPE = N_PAD // NS
NBUF = 8
N_ITER = 2


def _mp_sc(h_hbm, src_hbm, dst_hbm, zeros_hbm, tv_hbm, ix_hbm, out_hbm, *scratch):
    src_v, dst_v = scratch[0], scratch[1]
    rows = scratch[2:2 + NBUF]
    acc_v, tbl_v, tv_v, ix_v = scratch[2 + NBUF:6 + NBUF]
    tbl_s, acc_s = scratch[6 + NBUF], scratch[7 + NBUF]
    sems = scratch[8 + NBUF:]
    c = lax.axis_index("c")
    s = lax.axis_index("s")
    stripe = pl.ds(s * STRIPE, STRIPE)
    pltpu.sync_copy(src_hbm.at[s], src_v)
    pltpu.sync_copy(dst_hbm.at[s], dst_v)
    pltpu.sync_copy(tv_hbm, tv_v)
    pltpu.sync_copy(ix_hbm, ix_v)
    pltpu.sync_copy(h_hbm.at[c, stripe], tbl_v)
    pltpu.sync_copy(tbl_v, tbl_s.at[stripe])

    def gather(g, b):
        return pltpu.make_async_copy(tbl_s.at[src_v.at[g]], rows[b], sems[b])

    for t in range(N_ITER):
        pltpu.sync_copy(zeros_hbm.at[stripe], acc_v)
        pltpu.sync_copy(acc_v, acc_s.at[stripe])
        plsc.subcore_barrier()

        for b in range(NBUF):
            gather(b, b).start()

        def body(i, _):
            for b in range(NBUF):
                g = i * NBUF + b
                gather(g, b).wait()
                pltpu.sync_copy(rows[b], acc_s.at[dst_v.at[g]], add=True)

                @pl.when(g < G - NBUF)
                def _fire():
                    gather(g + NBUF, b).start()
            return _

        lax.fori_loop(0, G // NBUF, body, None)
        plsc.subcore_barrier()

        pltpu.sync_copy(acc_s.at[stripe], acc_v)
        pltpu.sync_copy(tbl_s.at[stripe], tbl_v)
        tv0, tv1, tv2, tv3 = tv_v[0], tv_v[1], tv_v[2], tv_v[3]
        bvec = tv_v[4]

        _dn = lax.GatherDimensionNumbers(
            offset_dims=(), collapsed_slice_dims=(0,), start_index_map=(0,))

        def take16(x, idx):
            return lax.gather(x, idx.reshape(16, 1), _dn, slice_sizes=(1,),
                              mode=lax.GatherScatterMode.PROMISE_IN_BOUNDS)

        b0, b1, b2, b3 = ix_v[0], ix_v[1], ix_v[2], ix_v[3]

        def upd(r, _):
            S = acc_v[r]
            gv0 = take16(S, b0)
            gv1 = take16(S, b1)
            gv2 = take16(S, b2)
            gv3 = take16(S, b3)
            q = 1.0 / jnp.maximum(gv3, 1.0)
            mixed = gv0 * tv0 + gv1 * tv1 + gv2 * tv2 + gv3 * tv3
            tbl_v[r] = tbl_v[r] + q * mixed + bvec
            return _

        lax.fori_loop(0, STRIPE, upd, None)
        if t < N_ITER - 1:
            pltpu.sync_copy(tbl_v, tbl_s.at[stripe])
            plsc.subcore_barrier()

    pltpu.sync_copy(tbl_v, out_hbm.at[c, stripe])


@functools.cache
def _build_mp():
    return pl.kernel(
        _mp_sc,
        mesh=plsc.VectorSubcoreMesh(core_axis_name="c", subcore_axis_name="s"),
        compiler_params=pltpu.CompilerParams(use_tc_tiling_on_sc=False),
        out_type=jax.ShapeDtypeStruct((NC, N_PAD, ROW), jnp.float32),
        scratch_types=(
            [pltpu.VMEM((G, GRP), jnp.int32),
             pltpu.VMEM((G, GRP), jnp.int32)]
            + [pltpu.VMEM((GRP, ROW), jnp.float32) for _ in range(NBUF)]
            + [pltpu.VMEM((STRIPE, ROW), jnp.float32),
               pltpu.VMEM((STRIPE, ROW), jnp.float32),
               pltpu.VMEM((8, 16), jnp.float32),
               pltpu.VMEM((8, 16), jnp.int32),
               pltpu.VMEM_SHARED((N_PAD, ROW), jnp.float32),
               pltpu.VMEM_SHARED((N_PAD, ROW), jnp.float32)]
            + [pltpu.SemaphoreType.DMA for _ in range(NBUF)]
        ),
    )


def kernel(positions, bonds, W_msg, b_msg, W_upd, b_upd):
    src = jnp.concatenate([bonds[:, 0], bonds[:, 1]])
    dst = jnp.concatenate([bonds[:, 1], bonds[:, 0]])
    pad = jnp.full((E_PAD - src.shape[0],), N_ATOMS, jnp.int32)
    src = jnp.concatenate([src, pad]).reshape(NS, G, GRP)
    dst = jnp.concatenate([dst, pad]).reshape(NS, G, GRP)

    M = W_upd @ W_msg
    v = W_upd @ b_msg
    t4 = jnp.zeros((4, 4), jnp.float32).at[:3, :3].set(M.T).at[3, :3].set(v)
    tv = jnp.zeros((8, 16), jnp.float32)
    tv = tv.at[:4, :].set(jnp.tile(t4, (1, 4)))
    tv = tv.at[4, :].set(jnp.tile(
        jnp.concatenate([b_upd, jnp.zeros((1,), jnp.float32)]), 4))

    hp = jnp.concatenate(
        [positions.transpose(1, 0, 2),
         jnp.ones((N_ATOMS, BATCH, 1), jnp.float32)], axis=2
    ).reshape(N_ATOMS, BATCH, 4)
    h = jnp.zeros((NC, N_PAD, ROW), jnp.float32)
    h = h.at[0, :N_ATOMS].set(hp[:, :4].reshape(N_ATOMS, ROW))
    h = h.at[1, :N_ATOMS].set(hp[:, 4:].reshape(N_ATOMS, ROW))

    zeros = jnp.zeros((N_PAD, ROW), jnp.float32)
    lane = jnp.arange(16, dtype=jnp.int32)
    ix = jnp.zeros((8, 16), jnp.int32)
    for cc in range(4):
        ix = ix.at[cc].set((lane // 4) * 4 + cc)
    out = _build_mp()(h, src, dst, zeros, tv, ix)

    o = out[:, :N_ATOMS].reshape(NC, N_ATOMS, 4, 4)[:, :, :, :3]
    return o.transpose(0, 2, 1, 3).reshape(BATCH, N_ATOMS, 3)
